# baseline (speedup 1.0000x reference)
import functools
import math

import jax
import jax.numpy as jnp
from jax import lax
from jax.experimental import pallas as pl
from jax.experimental.pallas import tpu as pltpu
from jax.experimental.pallas import tpu_sc as plsc

D_MODEL = 64
SCALE = math.sqrt(D_MODEL)

NC = 2
NS = 16
NW = NC * NS
SB = 128


def _make_kernel(B):
    assert B % (NW * SB) == 0
    nstep = B // (NW * SB)
    mesh = plsc.VectorSubcoreMesh(
        core_axis_name="c", subcore_axis_name="s", num_cores=NC,
        num_subcores=NS)

    @functools.partial(
        pl.kernel,
        out_type=jax.ShapeDtypeStruct((B, D_MODEL), jnp.float32),
        mesh=mesh,
        scratch_types=[
            pltpu.VMEM((nstep, SB), jnp.int32),
            pltpu.VMEM((SB, D_MODEL), jnp.float32),
            pltpu.SemaphoreType.DMA,
        ],
        compiler_params=pltpu.CompilerParams(use_tc_tiling_on_sc=False),
    )
    def emb_kernel(idx_hbm, table_hbm, out_hbm, idx_v, rows_v, gsem):
        wid = lax.axis_index("s") * NC + lax.axis_index("c")
        pltpu.sync_copy(idx_hbm.at[wid], idx_v)
        base = wid * (nstep * SB)

        @pl.loop(0, nstep)
        def step(j):
            pltpu.async_copy(table_hbm.at[idx_v.at[j]], rows_v, gsem).wait()

            @pl.loop(0, SB)
            def srow(r):
                for c in range(D_MODEL // 16):
                    sl = (r, pl.ds(c * 16, 16))
                    rows_v[sl] = rows_v[sl] * SCALE

            pltpu.sync_copy(rows_v, out_hbm.at[pl.ds(base + j * SB, SB)])

    return emb_kernel


def kernel(x, table):
    B = x.size
    idx = x.reshape(NW, B // (NW * SB), SB)
    out = _make_kernel(B)(idx, table)
    return out.reshape(*x.shape, D_MODEL)


# speedup vs baseline: 1.2116x; 1.2116x over previous
import functools
import math

import jax
import jax.numpy as jnp
from jax import lax
from jax.experimental import pallas as pl
from jax.experimental.pallas import tpu as pltpu
from jax.experimental.pallas import tpu_sc as plsc

D_MODEL = 64
SCALE = math.sqrt(D_MODEL)

NC = 2
NS = 16
NW = NC * NS
SB = 128
NBUF = 4


def _make_kernel(B):
    assert B % (NW * SB) == 0
    nstep = B // (NW * SB)
    assert nstep % NBUF == 0
    ngrp = nstep // NBUF
    mesh = plsc.VectorSubcoreMesh(
        core_axis_name="c", subcore_axis_name="s", num_cores=NC,
        num_subcores=NS)

    @functools.partial(
        pl.kernel,
        out_type=jax.ShapeDtypeStruct((B, D_MODEL), jnp.float32),
        mesh=mesh,
        scratch_types=[
            pltpu.VMEM((nstep, SB), jnp.int32),
            pltpu.VMEM((NBUF, SB, D_MODEL), jnp.float32),
            pltpu.VMEM((NBUF, SB, D_MODEL), jnp.float32),
            pltpu.SemaphoreType.DMA((NBUF,)),
            pltpu.SemaphoreType.DMA((NBUF,)),
        ],
        compiler_params=pltpu.CompilerParams(use_tc_tiling_on_sc=False),
    )
    def emb_kernel(idx_hbm, table_hbm, out_hbm, idx_v, in_v, sc_v,
                   gsem, osem):
        wid = lax.axis_index("s") * NC + lax.axis_index("c")
        pltpu.sync_copy(idx_hbm.at[wid], idx_v)
        base = wid * (nstep * SB)

        def fire_gather(j, b):
            pltpu.async_copy(table_hbm.at[idx_v.at[j]], in_v.at[b],
                             gsem.at[b])

        def wait_gather(b):
            pltpu.make_async_copy(table_hbm.at[idx_v.at[0]], in_v.at[b],
                                  gsem.at[b]).wait()

        def fire_out(j, b):
            pltpu.async_copy(sc_v.at[b], out_hbm.at[pl.ds(base + j * SB, SB)],
                             osem.at[b])

        def wait_out(b):
            pltpu.make_async_copy(sc_v.at[b],
                                  out_hbm.at[pl.ds(base, SB)],
                                  osem.at[b]).wait()

        def scale(b):
            @plsc.parallel_loop(0, SB, unroll=4)
            def srow(r):
                for c in range(D_MODEL // 16):
                    sl = (r, pl.ds(c * 16, 16))
                    sc_v[(b,) + sl] = in_v[(b,) + sl] * SCALE

        for b in range(NBUF):
            fire_gather(b, b)

        for b in range(NBUF):
            wait_gather(b)
            scale(b)
            fire_out(b, b)
            fire_gather(NBUF + b, b)

        @pl.loop(1, ngrp)
        def grp(g):
            j0 = g * NBUF
            for b in range(NBUF):
                wait_gather(b)
                wait_out(b)
                scale(b)
                fire_out(j0 + b, b)

                @pl.when(g < ngrp - 1)
                def _():
                    fire_gather(j0 + NBUF + b, b)

        for b in range(NBUF):
            wait_out(b)

    return emb_kernel


def kernel(x, table):
    B = x.size
    idx = x.reshape(NW, B // (NW * SB), SB)
    out = _make_kernel(B)(idx, table)
    return out.reshape(*x.shape, D_MODEL)
